# baseline (speedup 1.0000x reference)
import functools

import jax
import jax.numpy as jnp
from jax import lax
from jax.experimental import pallas as pl
from jax.experimental.pallas import tpu as pltpu
from jax.experimental.pallas import tpu_sc as plsc

B = 16384
D = 64
_MAIN = 999424
_TAIL = 1000000 - _MAIN

_info = plsc.get_sparse_core_info()
_NC, _NS, _L = _info.num_cores, _info.num_subcores, _info.num_lanes
_NW = _NC * _NS
_BPW = B // _NW
_CHUNK = 128
_NCHUNK = _BPW // _CHUNK
_HALF = _BPW // 2


def _mf_body(uid_hbm, iid_hbm, uemb_hbm, iemb_hbm, ubm_hbm, ibm_hbm,
             ut_hbm, it_hbm, gb_hbm,
             out_hbm,
             uidx_v, iidx_v, uhi_v, ihi_v, upr_v, ipr_v, urows_v, irows_v,
             ubrow_v, ibrow_v, bias_v, ut_v, it_v, out_v, gb_v, sem, sem2):
    wid = lax.axis_index("s") * _NC + lax.axis_index("c")
    base = wid * _BPW

    for j in range(_NCHUNK):
        pltpu.sync_copy(uid_hbm.at[pl.ds(base + j * _CHUNK, _CHUNK)], uidx_v.at[j])
        pltpu.sync_copy(iid_hbm.at[pl.ds(base + j * _CHUNK, _CHUNK)], iidx_v.at[j])
    pltpu.sync_copy(gb_hbm, gb_v)
    pltpu.sync_copy(ut_hbm, ut_v)
    pltpu.sync_copy(it_hbm, it_v)

    iota = lax.broadcasted_iota(jnp.int32, (_L,), 0)
    gbv = gb_v[...]
    for j in range(_NCHUNK):
        for k in range(_CHUNK // _L):
            sl = pl.ds(k * _L, _L)
            uv = uidx_v[j, sl]
            iv = iidx_v[j, sl]
            uhi_v[j, sl] = jnp.minimum(uv, _MAIN - 1) >> 7
            ihi_v[j, sl] = jnp.minimum(iv, _MAIN - 1) >> 7
            upr_v[j, sl] = uv >> 1
            ipr_v[j, sl] = iv >> 1

    for j in range(_NCHUNK):
        cu = pltpu.async_copy(ubm_hbm.at[uhi_v.at[j]], ubrow_v, sem2)
        ci = pltpu.async_copy(ibm_hbm.at[ihi_v.at[j]], ibrow_v, sem2)
        cu.wait()
        ci.wait()
        for k in range(_CHUNK // _L):
            sl = pl.ds(k * _L, _L)
            lr = iota + k * _L
            uv = uidx_v[j, sl]
            iv = iidx_v[j, sl]
            ubv = plsc.load_gather(ubrow_v, [lr, uv & 127])
            ibv = plsc.load_gather(ibrow_v, [lr, iv & 127])
            utail = plsc.load_gather(ut_v, [jnp.clip(uv - _MAIN, 0, _TAIL - 1)])
            itail = plsc.load_gather(it_v, [jnp.clip(iv - _MAIN, 0, _TAIL - 1)])
            ubv = jnp.where(uv >= _MAIN, utail, ubv)
            ibv = jnp.where(iv >= _MAIN, itail, ibv)
            bias_v[pl.ds(j * _CHUNK + k * _L, _L)] = ubv + ibv + gbv

    for h in range(2):
        copies = []
        for c in range(2):
            j = 2 * h + c
            sl = pl.ds(c * _CHUNK, _CHUNK)
            copies.append(pltpu.async_copy(uemb_hbm.at[upr_v.at[j]], urows_v.at[sl], sem))
            copies.append(pltpu.async_copy(iemb_hbm.at[ipr_v.at[j]], irows_v.at[sl], sem))
        for c in copies:
            c.wait()

        def group(g, carry, h=h):
            r0 = g * _L
            jj = h * _HALF + r0
            uvec = uidx_v[jj // _CHUNK, pl.ds(jj % _CHUNK, _L)]
            ivec = iidx_v[jj // _CHUNK, pl.ds(jj % _CHUNK, _L)]
            acc = bias_v[pl.ds(jj, _L)]
            for l in range(_L):
                r = r0 + l
                uoff = (uvec[l] & 1) * 64
                ioff = (ivec[l] & 1) * 64
                p = urows_v[r, pl.ds(uoff, _L)] * irows_v[r, pl.ds(ioff, _L)]
                for k in range(1, D // _L):
                    p = p + (urows_v[r, pl.ds(uoff + k * _L, _L)]
                             * irows_v[r, pl.ds(ioff + k * _L, _L)])
                s = jnp.sum(p)
                acc = jnp.where(iota == l, acc + s, acc)
            out_v[pl.ds(jj, _L)] = acc
            return carry

        lax.fori_loop(0, _HALF // _L, group, 0)

    pltpu.sync_copy(out_v, out_hbm.at[pl.ds(base, _BPW)])


_mf_sc = functools.partial(
    pl.kernel,
    out_type=jax.ShapeDtypeStruct((B,), jnp.float32),
    mesh=plsc.VectorSubcoreMesh(core_axis_name="c", subcore_axis_name="s"),
    compiler_params=pltpu.CompilerParams(needs_layout_passes=False, use_tc_tiling_on_sc=True),
    scratch_types=[
        pltpu.VMEM((_NCHUNK, _CHUNK), jnp.int32),
        pltpu.VMEM((_NCHUNK, _CHUNK), jnp.int32),
        pltpu.VMEM((_NCHUNK, _CHUNK), jnp.int32),
        pltpu.VMEM((_NCHUNK, _CHUNK), jnp.int32),
        pltpu.VMEM((_NCHUNK, _CHUNK), jnp.int32),
        pltpu.VMEM((_NCHUNK, _CHUNK), jnp.int32),
        pltpu.VMEM((_HALF, 128), jnp.float32),
        pltpu.VMEM((_HALF, 128), jnp.float32),
        pltpu.VMEM((_CHUNK, 128), jnp.float32),
        pltpu.VMEM((_CHUNK, 128), jnp.float32),
        pltpu.VMEM((_BPW,), jnp.float32),
        pltpu.VMEM((_TAIL,), jnp.float32),
        pltpu.VMEM((_TAIL,), jnp.float32),
        pltpu.VMEM((_BPW,), jnp.float32),
        pltpu.VMEM((_L,), jnp.float32),
        pltpu.SemaphoreType.DMA,
        pltpu.SemaphoreType.DMA,
    ],
)(_mf_body)


def kernel(user_ids, item_ids, user_emb, item_emb, user_bias, item_bias, global_bias):
    uid = user_ids.astype(jnp.int32)
    iid = item_ids.astype(jnp.int32)
    uemb2 = user_emb.reshape(500000, 128)
    iemb2 = item_emb.reshape(500000, 128)
    ubm = user_bias[:_MAIN].reshape(_MAIN // 128, 128)
    ibm = item_bias[:_MAIN].reshape(_MAIN // 128, 128)
    ut = user_bias[_MAIN:].reshape(-1)
    it = item_bias[_MAIN:].reshape(-1)
    gb = jnp.broadcast_to(global_bias.reshape(()), (_L,))
    return _mf_sc(uid, iid, uemb2, iemb2, ubm, ibm, ut, it, gb)


# speedup vs baseline: 1.0127x; 1.0127x over previous
import functools

import jax
import jax.numpy as jnp
from jax import lax
from jax.experimental import pallas as pl
from jax.experimental.pallas import tpu as pltpu
from jax.experimental.pallas import tpu_sc as plsc

B = 16384
D = 64

_info = plsc.get_sparse_core_info()
_NC, _NS, _L = _info.num_cores, _info.num_subcores, _info.num_lanes
_NW = _NC * _NS
_BPW = B // _NW
_CHUNK = 128
_NCHUNK = _BPW // _CHUNK


def _mf_body(uid_hbm, iid_hbm, uemb_hbm, iemb_hbm, bias_hbm, gb_hbm,
             out_hbm,
             uidx_v, iidx_v, urows_v, irows_v, ub_v, ib_v, out_v, gb_v, sem):
    wid = lax.axis_index("s") * _NC + lax.axis_index("c")
    base = wid * _BPW

    for j in range(_NCHUNK):
        pltpu.sync_copy(uid_hbm.at[pl.ds(base + j * _CHUNK, _CHUNK)], uidx_v.at[j])
        pltpu.sync_copy(iid_hbm.at[pl.ds(base + j * _CHUNK, _CHUNK)], iidx_v.at[j])
    pltpu.sync_copy(gb_hbm, gb_v)

    copies = []
    for j in range(_NCHUNK):
        sl = pl.ds(j * _CHUNK, _CHUNK)
        copies.append(pltpu.async_copy(uemb_hbm.at[uidx_v.at[j]], urows_v.at[sl], sem))
        copies.append(pltpu.async_copy(iemb_hbm.at[iidx_v.at[j]], irows_v.at[sl], sem))
        copies.append(pltpu.async_copy(bias_hbm.at[pl.ds(0, 1000000)].at[uidx_v.at[j]], ub_v.at[sl], sem))
        copies.append(pltpu.async_copy(bias_hbm.at[pl.ds(1000000, 1000000)].at[iidx_v.at[j]], ib_v.at[sl], sem))
    for c in copies:
        c.wait()

    gbv = gb_v[...]
    iota = lax.broadcasted_iota(jnp.int32, (_L,), 0)

    def group(g, carry):
        r0 = g * _L
        acc = ub_v[pl.ds(r0, _L)] + ib_v[pl.ds(r0, _L)] + gbv
        for l in range(_L):
            r = r0 + l
            p = urows_v[r, pl.ds(0, _L)] * irows_v[r, pl.ds(0, _L)]
            for k in range(1, D // _L):
                p = p + urows_v[r, pl.ds(k * _L, _L)] * irows_v[r, pl.ds(k * _L, _L)]
            s = jnp.sum(p)
            acc = jnp.where(iota == l, acc + s, acc)
        out_v[pl.ds(r0, _L)] = acc
        return carry

    lax.fori_loop(0, _BPW // _L, group, 0)
    pltpu.sync_copy(out_v, out_hbm.at[pl.ds(base, _BPW)])


_mf_sc = functools.partial(
    pl.kernel,
    out_type=jax.ShapeDtypeStruct((B,), jnp.float32),
    mesh=plsc.VectorSubcoreMesh(core_axis_name="c", subcore_axis_name="s"),
    compiler_params=pltpu.CompilerParams(needs_layout_passes=False, use_tc_tiling_on_sc=False),
    scratch_types=[
        pltpu.VMEM((_NCHUNK, _CHUNK), jnp.int32),
        pltpu.VMEM((_NCHUNK, _CHUNK), jnp.int32),
        pltpu.VMEM((_BPW, D), jnp.float32),
        pltpu.VMEM((_BPW, D), jnp.float32),
        pltpu.VMEM((_BPW,), jnp.float32),
        pltpu.VMEM((_BPW,), jnp.float32),
        pltpu.VMEM((_BPW,), jnp.float32),
        pltpu.VMEM((_L,), jnp.float32),
        pltpu.SemaphoreType.DMA,
    ],
)(_mf_body)


def kernel(user_ids, item_ids, user_emb, item_emb, user_bias, item_bias, global_bias):
    uid = user_ids.astype(jnp.int32)
    iid = item_ids.astype(jnp.int32)
    biases = jnp.concatenate([user_bias, item_bias], axis=0).reshape(-1)
    gb = jnp.broadcast_to(global_bias.reshape(()), (_L,))
    return _mf_sc(uid, iid, user_emb, item_emb, biases, gb)
